# baseline (speedup 1.0000x reference)
import functools

import jax
import jax.numpy as jnp
from jax import lax
from jax.experimental import pallas as pl
from jax.experimental.pallas import tpu as pltpu
from jax.experimental.pallas import tpu_sc as plsc

N = 10000
D = 128
E = 320000
NC = 2
NS = 16
NW = NC * NS
NP = 10240
EP = 327680
EPW = EP // NW
CH = 128
GW = EPW // CH
RT = NP // NS
GWA = 80
GWB = 80

_f32 = jnp.float32


@functools.cache
def _sc_kernels():
    mesh = plsc.VectorSubcoreMesh(
        core_axis_name="c", subcore_axis_name="s",
        num_cores=NC, num_subcores=NS)

    TOTCH = EP // CH

    @functools.partial(
        pl.kernel,
        mesh=mesh,
        out_type=jax.ShapeDtypeStruct((NC * NP,), _f32),
        scratch_types=[
            pltpu.VMEM((2, CH), jnp.int32),
            pltpu.VMEM((2, CH), jnp.int32),
            pltpu.VMEM((CH,), _f32),
            pltpu.VMEM((RT,), _f32),
            pltpu.SemaphoreType.DMA,
            pltpu.SemaphoreType.DMA,
            pltpu.VMEM_SHARED((NP,), _f32),
        ],
    )
    def deg_call(ed_hbm, deg_hbm, ev0, ev1, onesv, tmpv, isem0, isem1,
                 hist_sh):
        ci = lax.axis_index("c")
        si = lax.axis_index("s")
        wid = ci * NS + si
        evs = (ev0, ev1)
        isems = (isem0, isem1)
        for i in range(CH // 16):
            onesv[pl.ds(16 * i, 16)] = jnp.full((16,), 1.0, _f32)
        for i in range(RT // 16):
            tmpv[pl.ds(16 * i, 16)] = jnp.zeros((16,), _f32)
        pltpu.sync_copy(tmpv, hist_sh.at[pl.ds(si * RT, RT)])
        plsc.subcore_barrier()
        cbase = wid * GW
        pltpu.async_copy(ed_hbm.at[cbase], ev0, isem0)

        def pair(gout, carry):
            for b in range(2):
                g = gout * 2 + b
                ev, isem = evs[b], isems[b]
                nev, nisem = evs[1 - b], isems[1 - b]
                pltpu.make_async_copy(ed_hbm.at[cbase], ev, isem).wait()

                @pl.when(g + 1 < GW)
                def _():
                    pltpu.async_copy(ed_hbm.at[cbase + g + 1], nev, nisem)

                pltpu.sync_copy(onesv, hist_sh.at[ev.at[1]], add=True)
            return carry

        lax.fori_loop(0, GW // 2, pair, 0)
        plsc.subcore_barrier()
        pltpu.sync_copy(hist_sh.at[pl.ds(si * RT, RT)], tmpv)
        pltpu.sync_copy(tmpv, deg_hbm.at[pl.ds(ci * NP + si * RT, RT)])

    @functools.partial(
        pl.kernel,
        mesh=mesh,
        out_type=jax.ShapeDtypeStruct((NC * NP, D), _f32),
        scratch_types=[
            pltpu.VMEM((2, CH), jnp.int32),
            pltpu.VMEM((2, CH), jnp.int32),
            pltpu.VMEM((CH, D), _f32),
            pltpu.VMEM((CH, D), _f32),
            pltpu.SemaphoreType.DMA,
            pltpu.SemaphoreType.DMA,
            pltpu.VMEM_SHARED((NP, D), _f32),
        ],
    )
    def rows_call(y_hbm, ed_hbm, acc_hbm, ev0, ev1, rows0, rows1,
                  gsem0, gsem1, acc_sh):
        ci = lax.axis_index("c")
        si = lax.axis_index("s")
        evs = (ev0, ev1)
        rowss = (rows0, rows1)
        gsems = (gsem0, gsem1)

        def zrow(r, carry):
            for c in range(D // 16):
                rows0[r, pl.ds(16 * c, 16)] = jnp.zeros((16,), _f32)
            return carry

        lax.fori_loop(0, CH, zrow, 0)
        for k in range(RT // CH):
            pltpu.sync_copy(rows0, acc_sh.at[pl.ds(si * RT + k * CH, CH)])
        plsc.subcore_barrier()
        cbase = jnp.where(ci == 0, si * GWA, NS * GWA + si * GWB)
        gmine = jnp.where(ci == 0, GWA, GWB)
        pltpu.sync_copy(ed_hbm.at[cbase], ev0)
        pltpu.async_copy(y_hbm.at[ev0.at[0]], rows0, gsem0)

        def pair(gout, carry):
            for b in range(2):
                g = gout * 2 + b
                ev, rows, gsem = evs[b], rowss[b], gsems[b]
                nev, nrows, ngsem = evs[1 - b], rowss[1 - b], gsems[1 - b]
                last = g + 1 >= gmine

                @pl.when(jnp.logical_not(last))
                def _():
                    pltpu.sync_copy(ed_hbm.at[cbase + g + 1], nev)

                pltpu.make_async_copy(y_hbm.at[ev.at[0]], rows, gsem).wait()

                @pl.when(jnp.logical_not(last))
                def _():
                    pltpu.async_copy(y_hbm.at[nev.at[0]], nrows, ngsem)

                pltpu.sync_copy(rows, acc_sh.at[ev.at[1]], add=True)
            return carry

        lax.fori_loop(0, gmine // 2, pair, 0)
        plsc.subcore_barrier()
        for k in range(RT // CH):
            pltpu.sync_copy(acc_sh.at[pl.ds(si * RT + k * CH, CH)], rows0)
            pltpu.sync_copy(
                rows0, acc_hbm.at[pl.ds(ci * NP + si * RT + k * CH, CH)])

    @functools.partial(
        pl.kernel,
        mesh=mesh,
        out_type=jax.ShapeDtypeStruct((NC * NP,), _f32),
        scratch_types=[
            pltpu.VMEM((2, CH), jnp.int32),
            pltpu.VMEM((2, CH), jnp.int32),
            pltpu.VMEM((CH,), _f32),
            pltpu.VMEM((CH,), _f32),
            pltpu.VMEM((RT,), _f32),
            pltpu.SemaphoreType.DMA,
            pltpu.SemaphoreType.DMA,
            pltpu.VMEM_SHARED((NP,), _f32),
        ],
    )
    def scal_call(vy_hbm, ed_hbm, s_hbm, ev0, ev1, val0, val1, tmpv,
                  gsem0, gsem1, s_sh):
        ci = lax.axis_index("c")
        si = lax.axis_index("s")
        wid = ci * NS + si
        evs = (ev0, ev1)
        vals = (val0, val1)
        gsems = (gsem0, gsem1)
        for i in range(RT // 16):
            tmpv[pl.ds(16 * i, 16)] = jnp.zeros((16,), _f32)
        pltpu.sync_copy(tmpv, s_sh.at[pl.ds(si * RT, RT)])
        plsc.subcore_barrier()
        cbase = wid * GW
        pltpu.sync_copy(ed_hbm.at[cbase], ev0)
        pltpu.async_copy(vy_hbm.at[ev0.at[0]], val0, gsem0)

        def pair(gout, carry):
            for b in range(2):
                g = gout * 2 + b
                ev, val, gsem = evs[b], vals[b], gsems[b]
                nev, nval, ngsem = evs[1 - b], vals[1 - b], gsems[1 - b]
                last = (b == 1) & (gout == GW // 2 - 1)

                @pl.when(jnp.logical_not(last))
                def _():
                    pltpu.sync_copy(ed_hbm.at[cbase + g + 1], nev)

                pltpu.make_async_copy(vy_hbm.at[ev.at[0]], val, gsem).wait()

                @pl.when(jnp.logical_not(last))
                def _():
                    pltpu.async_copy(vy_hbm.at[nev.at[0]], nval, ngsem)

                pltpu.sync_copy(val, s_sh.at[ev.at[1]], add=True)
            return carry

        lax.fori_loop(0, GW // 2, pair, 0)
        plsc.subcore_barrier()
        pltpu.sync_copy(s_sh.at[pl.ds(si * RT, RT)], tmpv)
        pltpu.sync_copy(tmpv, s_hbm.at[pl.ds(ci * NP + si * RT, RT)])

    return deg_call, rows_call, scal_call


BR = 1000


def _mm_body(x_ref, w_ref, xw_ref):
    xw_ref[...] = jnp.dot(x_ref[...], w_ref[...], preferred_element_type=_f32)


def _prep_body(deg_ref, xw_ref, y_ref, dinv_ref):
    d = deg_ref[...]
    dt = jnp.sum(d, axis=1) + 1.0
    di = 1.0 / jnp.sqrt(dt)
    y_ref[...] = xw_ref[...] * di[:, None]
    dinv_ref[...] = di[:, None]


def _mid_body(acc_ref, y_ref, dinv_ref, b1_ref, w2_ref, wfc_ref, vy_ref):
    a = acc_ref[0] + acc_ref[1] + y_ref[...]
    di = dinv_ref[...]
    h1 = jnp.maximum(a * di + b1_ref[...][None, :], 0.0)
    hw = jnp.dot(h1, w2_ref[...], preferred_element_type=_f32)
    v = jnp.dot(hw, wfc_ref[...], preferred_element_type=_f32)
    vy_ref[...] = v * di


def _final_body(s_ref, vy_ref, dinv_ref, b2_ref, wfc_ref, bfc_ref, out_ref):
    st = jnp.sum(s_ref[...], axis=1, keepdims=True)
    c = jnp.dot(b2_ref[...][None, :], wfc_ref[...],
                preferred_element_type=_f32) + bfc_ref[...]
    out_ref[...] = dinv_ref[...] * (st + vy_ref[...]) + c


def _mm_call(x, W1):
    return pl.pallas_call(
        _mm_body,
        grid=(N // BR,),
        in_specs=[
            pl.BlockSpec((BR, D), lambda i: (i, 0)),
            pl.BlockSpec((D, D), lambda i: (0, 0)),
        ],
        out_specs=pl.BlockSpec((BR, D), lambda i: (i, 0)),
        out_shape=jax.ShapeDtypeStruct((N, D), _f32),
    )(x, W1)


def _prep_call(deg2, xw):
    return pl.pallas_call(
        _prep_body,
        grid=(N // BR,),
        in_specs=[
            pl.BlockSpec((BR, 2), lambda i: (i, 0)),
            pl.BlockSpec((BR, D), lambda i: (i, 0)),
        ],
        out_specs=[
            pl.BlockSpec((BR, D), lambda i: (i, 0)),
            pl.BlockSpec((BR, 1), lambda i: (i, 0)),
        ],
        out_shape=[
            jax.ShapeDtypeStruct((N, D), _f32),
            jax.ShapeDtypeStruct((N, 1), _f32),
        ],
    )(deg2, xw)


def _mid_call(acc, y, dinv, b1, W2, Wfc):
    return pl.pallas_call(
        _mid_body,
        grid=(N // BR,),
        in_specs=[
            pl.BlockSpec((2, BR, D), lambda i: (0, i, 0)),
            pl.BlockSpec((BR, D), lambda i: (i, 0)),
            pl.BlockSpec((BR, 1), lambda i: (i, 0)),
            pl.BlockSpec((D,), lambda i: (0,)),
            pl.BlockSpec((D, D), lambda i: (0, 0)),
            pl.BlockSpec((D, 1), lambda i: (0, 0)),
        ],
        out_specs=pl.BlockSpec((BR, 1), lambda i: (i, 0)),
        out_shape=jax.ShapeDtypeStruct((N, 1), _f32),
    )(acc, y, dinv, b1, W2, Wfc)


def _final_call(s2, vy, dinv, b2, Wfc, bfc11):
    return pl.pallas_call(
        _final_body,
        grid=(N // BR,),
        in_specs=[
            pl.BlockSpec((BR, 2), lambda i: (i, 0)),
            pl.BlockSpec((BR, 1), lambda i: (i, 0)),
            pl.BlockSpec((BR, 1), lambda i: (i, 0)),
            pl.BlockSpec((D,), lambda i: (0,)),
            pl.BlockSpec((D, 1), lambda i: (0, 0)),
            pl.BlockSpec((1, 1), lambda i: (0, 0)),
        ],
        out_specs=pl.BlockSpec((BR, 1), lambda i: (i, 0)),
        out_shape=jax.ShapeDtypeStruct((N, 1), _f32),
    )(s2, vy, dinv, b2, Wfc, bfc11)


def kernel(x, edge_index, edge_attr, W1, b1, W2, b2, Wfc, bfc):
    deg_call, rows_call, scal_call = _sc_kernels()
    src = edge_index[0].astype(jnp.int32)
    dst = edge_index[1].astype(jnp.int32)
    pad = EP - E
    src_p = jnp.concatenate(
        [src, jnp.arange(pad, dtype=jnp.int32) % N])
    dst_p = jnp.concatenate(
        [dst, N + (jnp.arange(pad, dtype=jnp.int32) % (NP - N))])
    ed_p = jnp.stack(
        [src_p.reshape(EP // CH, CH), dst_p.reshape(EP // CH, CH)], axis=1)

    xw = _mm_call(x, W1)
    deg2 = deg_call(ed_p).reshape(NC, NP)[:, :N].T
    y, dinv = _prep_call(deg2, xw)
    acc = rows_call(y, ed_p).reshape(NC, NP, D)[:, :N]
    vy = _mid_call(acc, y, dinv, b1, W2, Wfc)
    s2 = scal_call(vy.reshape(N), ed_p).reshape(NC, NP)[:, :N].T
    return _final_call(s2, vy, dinv, b2, Wfc, bfc.reshape(1, 1))


# speedup vs baseline: 1.0905x; 1.0905x over previous
import functools

import jax
import jax.numpy as jnp
from jax import lax
from jax.experimental import pallas as pl
from jax.experimental.pallas import tpu as pltpu
from jax.experimental.pallas import tpu_sc as plsc

N = 10000
D = 128
E = 320000
NC = 2
NS = 16
NW = NC * NS
NP = 10240
EP = 327680
EPW = EP // NW
CH = 128
GW = EPW // CH
RT = NP // NS
GWA = 80
GWB = 80

_f32 = jnp.float32


@functools.cache
def _sc_kernels():
    mesh = plsc.VectorSubcoreMesh(
        core_axis_name="c", subcore_axis_name="s",
        num_cores=NC, num_subcores=NS)

    TOTCH = EP // CH

    @functools.partial(
        pl.kernel,
        mesh=mesh,
        out_type=jax.ShapeDtypeStruct((NC * NP,), _f32),
        scratch_types=[
            pltpu.VMEM((2, CH), jnp.int32),
            pltpu.VMEM((2, CH), jnp.int32),
            pltpu.VMEM((CH,), _f32),
            pltpu.VMEM((RT,), _f32),
            pltpu.SemaphoreType.DMA,
            pltpu.SemaphoreType.DMA,
            pltpu.VMEM_SHARED((NP,), _f32),
        ],
    )
    def deg_call(ed_hbm, deg_hbm, ev0, ev1, onesv, tmpv, isem0, isem1,
                 hist_sh):
        ci = lax.axis_index("c")
        si = lax.axis_index("s")
        wid = ci * NS + si
        evs = (ev0, ev1)
        isems = (isem0, isem1)
        for i in range(CH // 16):
            onesv[pl.ds(16 * i, 16)] = jnp.full((16,), 1.0, _f32)
        for i in range(RT // 16):
            tmpv[pl.ds(16 * i, 16)] = jnp.zeros((16,), _f32)
        pltpu.sync_copy(tmpv, hist_sh.at[pl.ds(si * RT, RT)])
        plsc.subcore_barrier()
        cbase = wid * GW
        pltpu.async_copy(ed_hbm.at[cbase], ev0, isem0)

        def pair(gout, carry):
            for b in range(2):
                g = gout * 2 + b
                ev, isem = evs[b], isems[b]
                nev, nisem = evs[1 - b], isems[1 - b]
                pltpu.make_async_copy(ed_hbm.at[cbase], ev, isem).wait()

                @pl.when(g + 1 < GW)
                def _():
                    pltpu.async_copy(ed_hbm.at[cbase + g + 1], nev, nisem)

                pltpu.sync_copy(onesv, hist_sh.at[ev.at[1]], add=True)
            return carry

        lax.fori_loop(0, GW // 2, pair, 0)
        plsc.subcore_barrier()
        pltpu.sync_copy(hist_sh.at[pl.ds(si * RT, RT)], tmpv)
        pltpu.sync_copy(tmpv, deg_hbm.at[pl.ds(ci * NP + si * RT, RT)])

    @functools.partial(
        pl.kernel,
        mesh=mesh,
        out_type=jax.ShapeDtypeStruct((NC * NP, D), _f32),
        scratch_types=[
            pltpu.VMEM((2, CH), jnp.int32),
            pltpu.VMEM((2, CH), jnp.int32),
            pltpu.VMEM((CH, D), _f32),
            pltpu.VMEM((CH, D), _f32),
            pltpu.SemaphoreType.DMA,
            pltpu.SemaphoreType.DMA,
            pltpu.VMEM_SHARED((NP, D), _f32),
        ],
    )
    def rows_call(y_hbm, ed_hbm, acc_hbm, ev0, ev1, rw0, rw1,
                  gs0, gs1, acc_sh):
        ci = lax.axis_index("c")
        si = lax.axis_index("s")
        wid = ci * NS + si
        evs = (ev0, ev1)
        rowss = (rw0, rw1)
        gsems = (gs0, gs1)
        rows0 = rw0

        def zrow(r, carry):
            for c in range(D // 16):
                rows0[r, pl.ds(16 * c, 16)] = jnp.zeros((16,), _f32)
            return carry

        lax.fori_loop(0, CH, zrow, 0)
        for k in range(RT // CH):
            pltpu.sync_copy(rows0, acc_sh.at[pl.ds(si * RT + k * CH, CH)])
        plsc.subcore_barrier()
        cbase = wid * GW
        pltpu.sync_copy(ed_hbm.at[cbase], ev0)
        pltpu.async_copy(y_hbm.at[ev0.at[0]], rw0, gs0)

        def pair(gout, carry):
            for b in range(2):
                g = gout * 2 + b
                ev, rows, gsem = evs[b], rowss[b], gsems[b]
                nev, nrows, ngsem = evs[1 - b], rowss[1 - b], gsems[1 - b]

                @pl.when(g + 1 < GW)
                def _():
                    pltpu.sync_copy(ed_hbm.at[cbase + g + 1], nev)

                pltpu.make_async_copy(y_hbm.at[ev.at[0]], rows, gsem).wait()

                @pl.when(g + 1 < GW)
                def _():
                    pltpu.async_copy(y_hbm.at[nev.at[0]], nrows, ngsem)

                pltpu.sync_copy(rows, acc_sh.at[ev.at[1]], add=True)
            return carry

        lax.fori_loop(0, GW // 2, pair, 0)
        plsc.subcore_barrier()
        for k in range(RT // CH):
            pltpu.sync_copy(acc_sh.at[pl.ds(si * RT + k * CH, CH)], rows0)
            pltpu.sync_copy(
                rows0, acc_hbm.at[pl.ds(ci * NP + si * RT + k * CH, CH)])

    @functools.partial(
        pl.kernel,
        mesh=mesh,
        out_type=jax.ShapeDtypeStruct((NC * NP,), _f32),
        scratch_types=[
            pltpu.VMEM((2, CH), jnp.int32),
            pltpu.VMEM((2, CH), jnp.int32),
            pltpu.VMEM((2, CH), jnp.int32),
            pltpu.VMEM((2, CH), jnp.int32),
            pltpu.VMEM((CH,), _f32),
            pltpu.VMEM((CH,), _f32),
            pltpu.VMEM((CH,), _f32),
            pltpu.VMEM((CH,), _f32),
            pltpu.SemaphoreType.DMA,
            pltpu.SemaphoreType.DMA,
            pltpu.SemaphoreType.DMA,
            pltpu.SemaphoreType.DMA,
            pltpu.VMEM((RT,), _f32),
            pltpu.VMEM_SHARED((NP,), _f32),
        ],
    )
    def scal_call(vy_hbm, ed_hbm, s_hbm, ev0, ev1, ev2, ev3,
                  vl0, vl1, vl2, vl3, gs0, gs1, gs2, gs3, tmpv, s_sh):
        ci = lax.axis_index("c")
        si = lax.axis_index("s")
        wid = ci * NS + si
        evs = (ev0, ev1, ev2, ev3)
        vals = (vl0, vl1, vl2, vl3)
        gsems = (gs0, gs1, gs2, gs3)
        for i in range(RT // 16):
            tmpv[pl.ds(16 * i, 16)] = jnp.zeros((16,), _f32)
        pltpu.sync_copy(tmpv, s_sh.at[pl.ds(si * RT, RT)])
        plsc.subcore_barrier()
        cbase = wid * GW
        for p in range(2):
            pltpu.sync_copy(ed_hbm.at[cbase + p], evs[p])
            pltpu.async_copy(vy_hbm.at[evs[p].at[0]], vals[p], gsems[p])

        def quad(gout, carry):
            for b in range(4):
                g = gout * 4 + b
                ev, val, gsem = evs[b], vals[b], gsems[b]
                b2 = (b + 2) % 4
                nev, nval, ngsem = evs[b2], vals[b2], gsems[b2]

                @pl.when(g + 2 < GW)
                def _():
                    pltpu.sync_copy(ed_hbm.at[cbase + g + 2], nev)
                    pltpu.async_copy(vy_hbm.at[nev.at[0]], nval, ngsem)

                pltpu.make_async_copy(vy_hbm.at[ev.at[0]], val, gsem).wait()
                pltpu.sync_copy(val, s_sh.at[ev.at[1]], add=True)
            return carry

        lax.fori_loop(0, GW // 4, quad, 0)
        plsc.subcore_barrier()
        pltpu.sync_copy(s_sh.at[pl.ds(si * RT, RT)], tmpv)
        pltpu.sync_copy(tmpv, s_hbm.at[pl.ds(ci * NP + si * RT, RT)])

    return deg_call, rows_call, scal_call


BR = 1000


def _mm_body(x_ref, w_ref, xw_ref):
    xw_ref[...] = jnp.dot(x_ref[...].astype(jnp.bfloat16),
                          w_ref[...].astype(jnp.bfloat16),
                          preferred_element_type=_f32)


def _prep_body(deg_ref, xw_ref, y_ref, dinv_ref):
    d = deg_ref[...]
    dt = jnp.sum(d, axis=1) + 1.0
    di = 1.0 / jnp.sqrt(dt)
    y_ref[...] = xw_ref[...] * di[:, None]
    dinv_ref[...] = di[:, None]


def _mid_body(acc_ref, y_ref, dinv_ref, b1_ref, w2_ref, wfc_ref, vy_ref):
    a = acc_ref[0] + acc_ref[1] + y_ref[...]
    di = dinv_ref[...]
    h1 = jnp.maximum(a * di + b1_ref[...][None, :], 0.0)
    hw = jnp.dot(h1.astype(jnp.bfloat16),
                 w2_ref[...].astype(jnp.bfloat16),
                 preferred_element_type=_f32)
    v = jnp.dot(hw.astype(jnp.bfloat16),
                wfc_ref[...].astype(jnp.bfloat16),
                preferred_element_type=_f32)
    vy_ref[...] = v * di


def _final_body(s_ref, vy_ref, dinv_ref, b2_ref, wfc_ref, bfc_ref, out_ref):
    st = jnp.sum(s_ref[...], axis=1, keepdims=True)
    c = jnp.dot(b2_ref[...][None, :], wfc_ref[...],
                preferred_element_type=_f32) + bfc_ref[...]
    out_ref[...] = dinv_ref[...] * (st + vy_ref[...]) + c


def _mm_call(x, W1):
    return pl.pallas_call(
        _mm_body,
        grid=(N // BR,),
        in_specs=[
            pl.BlockSpec((BR, D), lambda i: (i, 0)),
            pl.BlockSpec((D, D), lambda i: (0, 0)),
        ],
        out_specs=pl.BlockSpec((BR, D), lambda i: (i, 0)),
        out_shape=jax.ShapeDtypeStruct((N, D), _f32),
    )(x, W1)


def _prep_call(deg2, xw):
    return pl.pallas_call(
        _prep_body,
        grid=(N // BR,),
        in_specs=[
            pl.BlockSpec((BR, 2), lambda i: (i, 0)),
            pl.BlockSpec((BR, D), lambda i: (i, 0)),
        ],
        out_specs=[
            pl.BlockSpec((BR, D), lambda i: (i, 0)),
            pl.BlockSpec((BR, 1), lambda i: (i, 0)),
        ],
        out_shape=[
            jax.ShapeDtypeStruct((N, D), _f32),
            jax.ShapeDtypeStruct((N, 1), _f32),
        ],
    )(deg2, xw)


def _mid_call(acc, y, dinv, b1, W2, Wfc):
    return pl.pallas_call(
        _mid_body,
        grid=(N // BR,),
        in_specs=[
            pl.BlockSpec((2, BR, D), lambda i: (0, i, 0)),
            pl.BlockSpec((BR, D), lambda i: (i, 0)),
            pl.BlockSpec((BR, 1), lambda i: (i, 0)),
            pl.BlockSpec((D,), lambda i: (0,)),
            pl.BlockSpec((D, D), lambda i: (0, 0)),
            pl.BlockSpec((D, 1), lambda i: (0, 0)),
        ],
        out_specs=pl.BlockSpec((BR, 1), lambda i: (i, 0)),
        out_shape=jax.ShapeDtypeStruct((N, 1), _f32),
    )(acc, y, dinv, b1, W2, Wfc)


def _final_call(s2, vy, dinv, b2, Wfc, bfc11):
    return pl.pallas_call(
        _final_body,
        grid=(N // BR,),
        in_specs=[
            pl.BlockSpec((BR, 2), lambda i: (i, 0)),
            pl.BlockSpec((BR, 1), lambda i: (i, 0)),
            pl.BlockSpec((BR, 1), lambda i: (i, 0)),
            pl.BlockSpec((D,), lambda i: (0,)),
            pl.BlockSpec((D, 1), lambda i: (0, 0)),
            pl.BlockSpec((1, 1), lambda i: (0, 0)),
        ],
        out_specs=pl.BlockSpec((BR, 1), lambda i: (i, 0)),
        out_shape=jax.ShapeDtypeStruct((N, 1), _f32),
    )(s2, vy, dinv, b2, Wfc, bfc11)


def kernel(x, edge_index, edge_attr, W1, b1, W2, b2, Wfc, bfc):
    deg_call, rows_call, scal_call = _sc_kernels()
    src = edge_index[0].astype(jnp.int32)
    dst = edge_index[1].astype(jnp.int32)
    pad = EP - E
    src_p = jnp.concatenate(
        [src, jnp.arange(pad, dtype=jnp.int32) % N])
    dst_p = jnp.concatenate(
        [dst, N + (jnp.arange(pad, dtype=jnp.int32) % (NP - N))])
    ed_p = jnp.stack(
        [src_p.reshape(EP // CH, CH), dst_p.reshape(EP // CH, CH)], axis=1)

    xw = _mm_call(x, W1)
    deg2 = deg_call(ed_p).reshape(NC, NP)[:, :N].T
    y, dinv = _prep_call(deg2, xw)
    acc = rows_call(y, ed_p).reshape(NC, NP, D)[:, :N]
    vy = _mid_call(acc, y, dinv, b1, W2, Wfc)
    s2 = scal_call(vy.reshape(N), ed_p).reshape(NC, NP)[:, :N].T
    return _final_call(s2, vy, dinv, b2, Wfc, bfc.reshape(1, 1))
